# baseline (speedup 1.0000x reference)
import functools

import jax
import jax.numpy as jnp
from jax import lax
from jax.experimental import pallas as pl
from jax.experimental.pallas import tpu as pltpu
from jax.experimental.pallas import tpu_sc as plsc

N_NODES = 10000
N_EDGES = 320000
D = 128

NC = 2
NS = 16
NW = NC * NS
C = 64
E_W = N_EDGES // NW
NCH = 160
SLOTS = NCH * C
NB = 2
SLAB = 640
N_PAD = NS * SLAB


def _sc_agg_body(feat, src_r, dst_r, zrows, agg_out,
                 didx_all, sidx0, sidx1,
                 rows0, rows1,
                 gsem0, gsem1,
                 ssem0, ssem1,
                 isem0, isem1, agg_sh):
    sidx = [sidx0, sidx1]
    rows = [rows0, rows1]
    gsem = [gsem0, gsem1]
    ssem = [ssem0, ssem1]
    isem = [isem0, isem1]
    cid = lax.axis_index("c")
    sid = lax.axis_index("s")
    wid = cid * NS + sid

    pltpu.sync_copy(zrows, rows0)
    for z in range(SLAB // C):
        pltpu.sync_copy(rows0, agg_sh.at[pl.ds(sid * SLAB + z * C, C)])
    pltpu.sync_copy(dst_r.at[wid], didx_all)
    plsc.subcore_barrier()

    for b in range(NB):
        pltpu.sync_copy(src_r.at[wid, b], sidx[b])
        pltpu.async_copy(feat.at[sidx[b]], rows[b], gsem[b])

    def outer(g, carry):
        for b in range(NB):
            c = g * NB + b
            pltpu.make_async_copy(feat.at[sidx[b]], rows[b], gsem[b]).wait()
            pltpu.async_copy(rows[b], agg_sh.at[didx_all.at[c]], ssem[b],
                             add=True)

            @pl.when(c + NB < NCH)
            def _():
                pltpu.async_copy(src_r.at[wid, c + NB], sidx[b], isem[b])

            pltpu.make_async_copy(rows[b], agg_sh.at[didx_all.at[c]],
                                  ssem[b]).wait()

            @pl.when(c + NB < NCH)
            def _():
                pltpu.make_async_copy(src_r.at[wid, c + NB], sidx[b],
                                      isem[b]).wait()
                pltpu.async_copy(feat.at[sidx[b]], rows[b], gsem[b])

        return carry

    lax.fori_loop(0, NCH // NB, outer, 0)

    plsc.subcore_barrier()
    pltpu.sync_copy(agg_sh.at[pl.ds(sid * SLAB, SLAB)],
                    agg_out.at[cid, pl.ds(sid * SLAB, SLAB)])


def _make_sc_agg():
    mesh = plsc.VectorSubcoreMesh(core_axis_name="c", subcore_axis_name="s")
    return pl.kernel(
        _sc_agg_body,
        out_type=jax.ShapeDtypeStruct((NC, N_PAD, D), jnp.float32),
        mesh=mesh,
        scratch_types=(
            [pltpu.VMEM((NCH, C), jnp.int32)]
            + [pltpu.VMEM((C,), jnp.int32)] * NB
            + [pltpu.VMEM((C, D), jnp.float32)] * NB
            + [pltpu.SemaphoreType.DMA] * (3 * NB)
            + [pltpu.VMEM_SHARED((N_PAD, D), jnp.float32)]
        ),
    )


def _sc_deg_body(dst_r, zrows, ones_h, deg_out, didx_all, ones_v, ssem, deg_sh):
    cid = lax.axis_index("c")
    sid = lax.axis_index("s")
    wid = cid * NS + sid

    pltpu.sync_copy(zrows, ones_v)
    for z in range(SLAB // C):
        pltpu.sync_copy(ones_v, deg_sh.at[pl.ds(sid * SLAB + z * C, C)])
    pltpu.sync_copy(ones_h, ones_v)
    pltpu.sync_copy(dst_r.at[wid], didx_all)
    plsc.subcore_barrier()

    def fire(c, carry):
        pltpu.async_copy(ones_v, deg_sh.at[didx_all.at[c]], ssem, add=True)
        return carry

    lax.fori_loop(0, NCH, fire, 0)

    def drain(c, carry):
        pltpu.make_async_copy(ones_v, deg_sh.at[didx_all.at[c]], ssem).wait()
        return carry

    lax.fori_loop(0, NCH, drain, 0)

    plsc.subcore_barrier()
    pltpu.sync_copy(deg_sh.at[pl.ds(sid * SLAB, SLAB)],
                    deg_out.at[cid, pl.ds(sid * SLAB, SLAB)])


def _make_sc_deg():
    mesh = plsc.VectorSubcoreMesh(core_axis_name="c", subcore_axis_name="s")
    return pl.kernel(
        _sc_deg_body,
        out_type=jax.ShapeDtypeStruct((NC, N_PAD, D), jnp.float32),
        mesh=mesh,
        scratch_types=[
            pltpu.VMEM((NCH, C), jnp.int32),
            pltpu.VMEM((C, D), jnp.float32),
            pltpu.SemaphoreType.DMA,
            pltpu.VMEM_SHARED((N_PAD, D), jnp.float32),
        ],
    )


def _tc_dense_body(relu, agg_ref, deg_ref, x_ref, wl_ref, b_ref, wr_ref, o_ref):
    agg = agg_ref[0] + agg_ref[1]
    deg = deg_ref[0, :, 0:1] + deg_ref[1, :, 0:1]
    mean = agg * (1.0 / jnp.maximum(deg, 1.0))
    h = (jnp.dot(mean, wl_ref[...], preferred_element_type=jnp.float32)
         + b_ref[...]
         + jnp.dot(x_ref[...], wr_ref[...], preferred_element_type=jnp.float32))
    o_ref[...] = jnp.maximum(h, 0.0) if relu else h


def _make_tc_dense(relu, br=512):
    grid = (N_PAD // br,)
    return pl.pallas_call(
        functools.partial(_tc_dense_body, relu),
        grid=grid,
        in_specs=[
            pl.BlockSpec((NC, br, D), lambda i: (0, i, 0)),
            pl.BlockSpec((NC, br, D), lambda i: (0, i, 0)),
            pl.BlockSpec((br, D), lambda i: (i, 0)),
            pl.BlockSpec((D, D), lambda i: (0, 0)),
            pl.BlockSpec((1, D), lambda i: (0, 0)),
            pl.BlockSpec((D, D), lambda i: (0, 0)),
        ],
        out_specs=pl.BlockSpec((br, D), lambda i: (i, 0)),
        out_shape=jax.ShapeDtypeStruct((N_PAD, D), jnp.float32),
    )


def kernel(x, edge_index, W1l, b1l, W1r, W2l, b2l, W2r):
    src = edge_index[0].astype(jnp.int32)
    dst = edge_index[1].astype(jnp.int32)

    flatp = jnp.arange(NW * (SLOTS - E_W), dtype=jnp.int32)
    pad_src = (flatp % N_NODES).reshape(NW, SLOTS - E_W)
    pad_dst = (N_NODES + flatp % (N_PAD - N_NODES)).reshape(NW, SLOTS - E_W)
    src_r = jnp.concatenate(
        [src.reshape(NW, E_W), pad_src], axis=1).reshape(NW, NCH, C)
    dst_r = jnp.concatenate(
        [dst.reshape(NW, E_W), pad_dst], axis=1).reshape(NW, NCH, C)
    x_p = jnp.pad(x, ((0, N_PAD - N_NODES), (0, 0)))

    zrows = jnp.zeros((C, D), jnp.float32)
    ones_h = jnp.ones((C, D), jnp.float32)

    sc_deg = _make_sc_deg()
    sc_agg = _make_sc_agg()
    tc1 = _make_tc_dense(True)
    tc2 = _make_tc_dense(False)

    deg_p = sc_deg(dst_r, zrows, ones_h)
    agg1 = sc_agg(x_p, src_r, dst_r, zrows)
    h = tc1(agg1, deg_p, x_p, W1l.T, b1l.reshape(1, D), W1r.T)
    agg2 = sc_agg(h, src_r, dst_r, zrows)
    out = tc2(agg2, deg_p, h, W2l.T, b2l.reshape(1, D), W2r.T)
    return out[:N_NODES]


# speedup vs baseline: 1.1827x; 1.1827x over previous
import functools

import jax
import jax.numpy as jnp
from jax import lax
from jax.experimental import pallas as pl
from jax.experimental.pallas import tpu as pltpu
from jax.experimental.pallas import tpu_sc as plsc

N_NODES = 10000
N_EDGES = 320000
D = 128

NC = 2
NS = 16
NW = NC * NS
C = 128
E_W = N_EDGES // NW
NCH = 80
SLOTS = NCH * C
NB = 2
SLAB = 640
N_PAD = NS * SLAB


def _sc_agg_body(feat, src_r, dst_r, zrows, agg_out,
                 didx_all, sidx0, sidx1,
                 rows0, rows1,
                 gsem0, gsem1,
                 ssem0, ssem1,
                 isem0, isem1, agg_sh):
    sidx = [sidx0, sidx1]
    rows = [rows0, rows1]
    gsem = [gsem0, gsem1]
    ssem = [ssem0, ssem1]
    isem = [isem0, isem1]
    cid = lax.axis_index("c")
    sid = lax.axis_index("s")
    wid = cid * NS + sid

    pltpu.sync_copy(zrows, rows0)
    for z in range(SLAB // C):
        pltpu.sync_copy(rows0, agg_sh.at[pl.ds(sid * SLAB + z * C, C)])
    pltpu.sync_copy(dst_r.at[wid], didx_all)
    plsc.subcore_barrier()

    for b in range(NB):
        pltpu.sync_copy(src_r.at[wid, b], sidx[b])
        pltpu.async_copy(feat.at[sidx[b]], rows[b], gsem[b])

    def outer(g, carry):
        for b in range(NB):
            c = g * NB + b
            pltpu.make_async_copy(feat.at[sidx[b]], rows[b], gsem[b]).wait()
            pltpu.async_copy(rows[b], agg_sh.at[didx_all.at[c]], ssem[b],
                             add=True)

            @pl.when(c + NB < NCH)
            def _():
                pltpu.async_copy(src_r.at[wid, c + NB], sidx[b], isem[b])

            pltpu.make_async_copy(rows[b], agg_sh.at[didx_all.at[c]],
                                  ssem[b]).wait()

            @pl.when(c + NB < NCH)
            def _():
                pltpu.make_async_copy(src_r.at[wid, c + NB], sidx[b],
                                      isem[b]).wait()
                pltpu.async_copy(feat.at[sidx[b]], rows[b], gsem[b])

        return carry

    lax.fori_loop(0, NCH // NB, outer, 0)

    plsc.subcore_barrier()
    pltpu.sync_copy(agg_sh.at[pl.ds(sid * SLAB, SLAB)],
                    agg_out.at[cid, pl.ds(sid * SLAB, SLAB)])


def _make_sc_agg():
    mesh = plsc.VectorSubcoreMesh(core_axis_name="c", subcore_axis_name="s")
    return pl.kernel(
        _sc_agg_body,
        out_type=jax.ShapeDtypeStruct((NC, N_PAD, D), jnp.float32),
        mesh=mesh,
        scratch_types=(
            [pltpu.VMEM((NCH, C), jnp.int32)]
            + [pltpu.VMEM((C,), jnp.int32)] * NB
            + [pltpu.VMEM((C, D), jnp.float32)] * NB
            + [pltpu.SemaphoreType.DMA] * (3 * NB)
            + [pltpu.VMEM_SHARED((N_PAD, D), jnp.float32)]
        ),
    )


def _sc_deg_body(dst_r, zrows, ones_h, deg_out, didx_all, ones_v, ssem, deg_sh):
    cid = lax.axis_index("c")
    sid = lax.axis_index("s")
    wid = cid * NS + sid

    pltpu.sync_copy(zrows, ones_v)
    for z in range(SLAB // C):
        pltpu.sync_copy(ones_v, deg_sh.at[pl.ds(sid * SLAB + z * C, C)])
    pltpu.sync_copy(ones_h, ones_v)
    pltpu.sync_copy(dst_r.at[wid], didx_all)
    plsc.subcore_barrier()

    def fire(c, carry):
        pltpu.async_copy(ones_v, deg_sh.at[didx_all.at[c]], ssem, add=True)
        return carry

    lax.fori_loop(0, NCH, fire, 0)

    def drain(c, carry):
        pltpu.make_async_copy(ones_v, deg_sh.at[didx_all.at[c]], ssem).wait()
        return carry

    lax.fori_loop(0, NCH, drain, 0)

    plsc.subcore_barrier()
    pltpu.sync_copy(deg_sh.at[pl.ds(sid * SLAB, SLAB)],
                    deg_out.at[cid, pl.ds(sid * SLAB, SLAB)])


def _make_sc_deg():
    mesh = plsc.VectorSubcoreMesh(core_axis_name="c", subcore_axis_name="s")
    return pl.kernel(
        _sc_deg_body,
        out_type=jax.ShapeDtypeStruct((NC, N_PAD, D), jnp.float32),
        mesh=mesh,
        scratch_types=[
            pltpu.VMEM((NCH, C), jnp.int32),
            pltpu.VMEM((C, D), jnp.float32),
            pltpu.SemaphoreType.DMA,
            pltpu.VMEM_SHARED((N_PAD, D), jnp.float32),
        ],
    )


def _tc_dense_body(relu, agg_ref, deg_ref, x_ref, wl_ref, b_ref, wr_ref, o_ref):
    agg = agg_ref[0] + agg_ref[1]
    deg = deg_ref[0, :, 0:1] + deg_ref[1, :, 0:1]
    mean = agg * (1.0 / jnp.maximum(deg, 1.0))
    h = (jnp.dot(mean, wl_ref[...], preferred_element_type=jnp.float32)
         + b_ref[...]
         + jnp.dot(x_ref[...], wr_ref[...], preferred_element_type=jnp.float32))
    o_ref[...] = jnp.maximum(h, 0.0) if relu else h


def _make_tc_dense(relu, br=512):
    grid = (N_PAD // br,)
    return pl.pallas_call(
        functools.partial(_tc_dense_body, relu),
        grid=grid,
        in_specs=[
            pl.BlockSpec((NC, br, D), lambda i: (0, i, 0)),
            pl.BlockSpec((NC, br, D), lambda i: (0, i, 0)),
            pl.BlockSpec((br, D), lambda i: (i, 0)),
            pl.BlockSpec((D, D), lambda i: (0, 0)),
            pl.BlockSpec((1, D), lambda i: (0, 0)),
            pl.BlockSpec((D, D), lambda i: (0, 0)),
        ],
        out_specs=pl.BlockSpec((br, D), lambda i: (i, 0)),
        out_shape=jax.ShapeDtypeStruct((N_PAD, D), jnp.float32),
    )


def kernel(x, edge_index, W1l, b1l, W1r, W2l, b2l, W2r):
    src = edge_index[0].astype(jnp.int32)
    dst = edge_index[1].astype(jnp.int32)

    flatp = jnp.arange(NW * (SLOTS - E_W), dtype=jnp.int32)
    pad_src = (flatp % N_NODES).reshape(NW, SLOTS - E_W)
    pad_dst = (N_NODES + flatp % (N_PAD - N_NODES)).reshape(NW, SLOTS - E_W)
    src_r = jnp.concatenate(
        [src.reshape(NW, E_W), pad_src], axis=1).reshape(NW, NCH, C)
    dst_r = jnp.concatenate(
        [dst.reshape(NW, E_W), pad_dst], axis=1).reshape(NW, NCH, C)
    x_p = jnp.pad(x, ((0, N_PAD - N_NODES), (0, 0)))

    zrows = jnp.zeros((C, D), jnp.float32)
    ones_h = jnp.ones((C, D), jnp.float32)

    sc_deg = _make_sc_deg()
    sc_agg = _make_sc_agg()
    tc1 = _make_tc_dense(True)
    tc2 = _make_tc_dense(False)

    deg_p = sc_deg(dst_r, zrows, ones_h)
    agg1 = sc_agg(x_p, src_r, dst_r, zrows)
    h = tc1(agg1, deg_p, x_p, W1l.T, b1l.reshape(1, D), W1r.T)
    agg2 = sc_agg(h, src_r, dst_r, zrows)
    out = tc2(agg2, deg_p, h, W2l.T, b2l.reshape(1, D), W2r.T)
    return out[:N_NODES]


# speedup vs baseline: 1.2032x; 1.0173x over previous
import functools

import jax
import jax.numpy as jnp
from jax import lax
from jax.experimental import pallas as pl
from jax.experimental.pallas import tpu as pltpu
from jax.experimental.pallas import tpu_sc as plsc

N_NODES = 10000
N_EDGES = 320000
D = 128

NC = 2
NS = 16
NW = NC * NS
C = 128
E_W = N_EDGES // NW
NCH = 80
SLOTS = NCH * C
NB = 2
SLAB = 640
N_PAD = NS * SLAB


def _sc_agg_body(feat, src_r, dst_r, zrows, ones_h, agg_out, deg_out,
                 didx_all, sidx0, sidx1,
                 rows0, rows1,
                 gsem0, gsem1,
                 ssem0, ssem1,
                 isem0, isem1, agg_sh):
    sidx = [sidx0, sidx1]
    rows = [rows0, rows1]
    gsem = [gsem0, gsem1]
    ssem = [ssem0, ssem1]
    isem = [isem0, isem1]
    cid = lax.axis_index("c")
    sid = lax.axis_index("s")
    wid = cid * NS + sid
    with_deg = deg_out is not None

    pltpu.sync_copy(zrows, rows0)
    for z in range(SLAB // C):
        pltpu.sync_copy(rows0, agg_sh.at[pl.ds(sid * SLAB + z * C, C)])
    pltpu.sync_copy(dst_r.at[wid], didx_all)
    plsc.subcore_barrier()

    if with_deg:
        pltpu.sync_copy(ones_h, rows1)

        def fire(c, carry):
            pltpu.async_copy(rows1, agg_sh.at[didx_all.at[c]], ssem1,
                             add=True)
            return carry

        lax.fori_loop(0, NCH, fire, 0)

        def drain(c, carry):
            pltpu.make_async_copy(rows1, agg_sh.at[didx_all.at[c]],
                                  ssem1).wait()
            return carry

        lax.fori_loop(0, NCH, drain, 0)

        plsc.subcore_barrier()
        pltpu.sync_copy(agg_sh.at[pl.ds(sid * SLAB, SLAB)],
                        deg_out.at[cid, pl.ds(sid * SLAB, SLAB)])

        for z in range(SLAB // C):
            pltpu.sync_copy(rows0, agg_sh.at[pl.ds(sid * SLAB + z * C, C)])
        plsc.subcore_barrier()

    for b in range(NB):
        pltpu.sync_copy(src_r.at[wid, b], sidx[b])
        pltpu.async_copy(feat.at[sidx[b]], rows[b], gsem[b])

    def outer(g, carry):
        for b in range(NB):
            c = g * NB + b
            pltpu.make_async_copy(feat.at[sidx[b]], rows[b], gsem[b]).wait()
            pltpu.async_copy(rows[b], agg_sh.at[didx_all.at[c]], ssem[b],
                             add=True)

            @pl.when(c + NB < NCH)
            def _():
                pltpu.async_copy(src_r.at[wid, c + NB], sidx[b], isem[b])

            pltpu.make_async_copy(rows[b], agg_sh.at[didx_all.at[c]],
                                  ssem[b]).wait()

            @pl.when(c + NB < NCH)
            def _():
                pltpu.make_async_copy(src_r.at[wid, c + NB], sidx[b],
                                      isem[b]).wait()
                pltpu.async_copy(feat.at[sidx[b]], rows[b], gsem[b])

        return carry

    lax.fori_loop(0, NCH // NB, outer, 0)

    plsc.subcore_barrier()
    pltpu.sync_copy(agg_sh.at[pl.ds(sid * SLAB, SLAB)],
                    agg_out.at[cid, pl.ds(sid * SLAB, SLAB)])


def _agg_only_body(feat, src_r, dst_r, zrows, agg_out, *rest):
    return _sc_agg_body(feat, src_r, dst_r, zrows, None, agg_out, None, *rest)


def _make_sc_agg(with_deg):
    mesh = plsc.VectorSubcoreMesh(core_axis_name="c", subcore_axis_name="s")
    part = jax.ShapeDtypeStruct((NC, N_PAD, D), jnp.float32)
    return pl.kernel(
        _sc_agg_body if with_deg else _agg_only_body,
        out_type=(part, part) if with_deg else part,
        mesh=mesh,
        scratch_types=(
            [pltpu.VMEM((NCH, C), jnp.int32)]
            + [pltpu.VMEM((C,), jnp.int32)] * NB
            + [pltpu.VMEM((C, D), jnp.float32)] * NB
            + [pltpu.SemaphoreType.DMA] * (3 * NB)
            + [pltpu.VMEM_SHARED((N_PAD, D), jnp.float32)]
        ),
    )


def _tc_dense_body(relu, agg_ref, deg_ref, x_ref, wl_ref, b_ref, wr_ref, o_ref):
    agg = agg_ref[0] + agg_ref[1]
    deg = deg_ref[0, :, 0:1] + deg_ref[1, :, 0:1]
    mean = agg * (1.0 / jnp.maximum(deg, 1.0))
    h = (jnp.dot(mean, wl_ref[...], preferred_element_type=jnp.float32)
         + b_ref[...]
         + jnp.dot(x_ref[...], wr_ref[...], preferred_element_type=jnp.float32))
    o_ref[...] = jnp.maximum(h, 0.0) if relu else h


def _make_tc_dense(relu, br=512):
    grid = (N_PAD // br,)
    return pl.pallas_call(
        functools.partial(_tc_dense_body, relu),
        grid=grid,
        in_specs=[
            pl.BlockSpec((NC, br, D), lambda i: (0, i, 0)),
            pl.BlockSpec((NC, br, D), lambda i: (0, i, 0)),
            pl.BlockSpec((br, D), lambda i: (i, 0)),
            pl.BlockSpec((D, D), lambda i: (0, 0)),
            pl.BlockSpec((1, D), lambda i: (0, 0)),
            pl.BlockSpec((D, D), lambda i: (0, 0)),
        ],
        out_specs=pl.BlockSpec((br, D), lambda i: (i, 0)),
        out_shape=jax.ShapeDtypeStruct((N_PAD, D), jnp.float32),
    )


def kernel(x, edge_index, W1l, b1l, W1r, W2l, b2l, W2r):
    src = edge_index[0].astype(jnp.int32)
    dst = edge_index[1].astype(jnp.int32)

    flatp = jnp.arange(NW * (SLOTS - E_W), dtype=jnp.int32)
    pad_src = (flatp % N_NODES).reshape(NW, SLOTS - E_W)
    pad_dst = (N_NODES + flatp % (N_PAD - N_NODES)).reshape(NW, SLOTS - E_W)
    src_r = jnp.concatenate(
        [src.reshape(NW, E_W), pad_src], axis=1).reshape(NW, NCH, C)
    dst_r = jnp.concatenate(
        [dst.reshape(NW, E_W), pad_dst], axis=1).reshape(NW, NCH, C)
    x_p = jnp.pad(x, ((0, N_PAD - N_NODES), (0, 0)))

    zrows = jnp.zeros((C, D), jnp.float32)
    ones_h = jnp.ones((C, D), jnp.float32)

    sc_agg_deg = _make_sc_agg(True)
    sc_agg = _make_sc_agg(False)
    tc1 = _make_tc_dense(True)
    tc2 = _make_tc_dense(False)

    agg1, deg_p = sc_agg_deg(x_p, src_r, dst_r, zrows, ones_h)
    h = tc1(agg1, deg_p, x_p, W1l.T, b1l.reshape(1, D), W1r.T)
    agg2 = sc_agg(h, src_r, dst_r, zrows)
    out = tc2(agg2, deg_p, h, W2l.T, b2l.reshape(1, D), W2r.T)
    return out[:N_NODES]
